# baseline (speedup 1.0000x reference)
import functools

import jax
import jax.numpy as jnp
from jax import lax
from jax.experimental import pallas as pl
from jax.experimental.pallas import tpu as pltpu
from jax.experimental.pallas import tpu_sc as plsc

N_USERS = 50000
N_ITEMS = 50000
N_TOTAL = N_USERS + N_ITEMS
EMB_DIM = 32
HALF = 16
N_EDGES = 1600000
N_HOPS = 3

NS = 16
NCH = 784
E_PAD = NS * NCH * 128
N_PAD = 100096
ROWS_PER_TILE = N_PAD // NS

ERING = 8
MRING = 4

_mesh = plsc.VectorSubcoreMesh(core_axis_name="c", subcore_axis_name="s")


@functools.partial(
    pl.kernel,
    mesh=_mesh,
    out_type=jax.ShapeDtypeStruct((2, N_PAD, HALF), jnp.float32),
    compiler_params=pltpu.CompilerParams(use_tc_tiling_on_sc=False),
    scratch_types=[
        pltpu.VMEM((ERING, 2, 128), jnp.int32),
        pltpu.VMEM((ERING, 128), jnp.float32),
        pltpu.VMEM((MRING, 128, HALF), jnp.float32),
        pltpu.SemaphoreType.DMA,
        pltpu.SemaphoreType.DMA,
        pltpu.SemaphoreType.DMA,
        pltpu.SemaphoreType.DMA,
        pltpu.SemaphoreType.DMA,
        pltpu.VMEM_SHARED((N_PAD, HALF), jnp.float32),
    ],
)
def _hop(tab_hbm, edge_hbm, eval_hbm, zeros_hbm, out_hbm,
         ebuf, vbuf, msg, esem0, esem1, gsem, ssem0, ssem1, acc_sh):
    c = lax.axis_index("c")
    s = lax.axis_index("s")
    tab = tab_hbm.at[c]

    pltpu.sync_copy(zeros_hbm, acc_sh.at[pl.ds(s * ROWS_PER_TILE, ROWS_PER_TILE)])
    plsc.subcore_barrier()

    cb = s * NCH

    def e_start(ch, sem):
        slot = lax.rem(ch, ERING)
        pltpu.async_copy(edge_hbm.at[cb + ch], ebuf.at[slot], sem)
        pltpu.async_copy(eval_hbm.at[cb + ch], vbuf.at[slot], sem)

    def e_wait(ch, sem):
        slot = lax.rem(ch, ERING)
        pltpu.make_async_copy(edge_hbm.at[cb + ch], ebuf.at[slot], sem).wait()
        pltpu.make_async_copy(eval_hbm.at[cb + ch], vbuf.at[slot], sem).wait()

    def g_copy(ch, sem):
        eslot = lax.rem(ch, ERING)
        mslot = lax.rem(ch, MRING)
        return pltpu.make_async_copy(
            tab.at[ebuf.at[eslot, 1]], msg.at[mslot], sem)

    def s_start(ch, sem):
        eslot = lax.rem(ch, ERING)
        mslot = lax.rem(ch, MRING)
        pltpu.async_copy(msg.at[mslot], acc_sh.at[ebuf.at[eslot, 0]], sem,
                         add=True)

    def s_wait(ch, sem):
        eslot = lax.rem(ch, ERING)
        mslot = lax.rem(ch, MRING)
        pltpu.make_async_copy(
            msg.at[mslot], acc_sh.at[ebuf.at[eslot, 0]], sem).wait()

    def scale(ch):
        eslot = lax.rem(ch, ERING)
        mslot = lax.rem(ch, MRING)
        for g in range(8):
            vv = vbuf[eslot, pl.ds(g * 16, 16)]
            for e in range(16):
                r = g * 16 + e
                msg[mslot, r, :] = msg[mslot, r, :] * vv[e]

    e_start(0, esem0)
    e_start(1, esem1)
    e_wait(0, esem0)
    e_start(2, esem0)
    e_wait(1, esem1)
    e_start(3, esem1)
    g_copy(0, gsem).start()

    def body(ch, _):
        par_even = lax.rem(ch, 2) == 0

        @pl.when(ch + 2 < NCH)
        def _():
            @pl.when(par_even)
            def _():
                e_wait(ch + 2, esem0)

            @pl.when(jnp.logical_not(par_even))
            def _():
                e_wait(ch + 2, esem1)

        @pl.when(ch + 4 < NCH)
        def _():
            @pl.when(par_even)
            def _():
                e_start(ch + 4, esem0)

            @pl.when(jnp.logical_not(par_even))
            def _():
                e_start(ch + 4, esem1)

        @pl.when(ch >= 2)
        def _():
            @pl.when(par_even)
            def _():
                s_wait(ch - 2, ssem0)

            @pl.when(jnp.logical_not(par_even))
            def _():
                s_wait(ch - 2, ssem1)

        g_copy(ch, gsem).wait()

        @pl.when(ch + 1 < NCH)
        def _():
            g_copy(ch + 1, gsem).start()

        scale(ch)

        @pl.when(par_even)
        def _():
            s_start(ch, ssem0)

        @pl.when(jnp.logical_not(par_even))
        def _():
            s_start(ch, ssem1)

        return 0

    lax.fori_loop(0, NCH, body, 0)

    s_wait(NCH - 2, ssem0)
    s_wait(NCH - 1, ssem1)

    plsc.subcore_barrier()
    sl = pl.ds(s * ROWS_PER_TILE, ROWS_PER_TILE)
    pltpu.sync_copy(acc_sh.at[sl], out_hbm.at[c, sl])


def kernel(user_embed, item_embed, edge_index, edge_vals):
    all_embed = jnp.concatenate([user_embed, item_embed], axis=0)
    all_embed = jnp.pad(all_embed, ((0, N_PAD - N_TOTAL), (0, 0)))
    tab = jnp.stack([all_embed[:, :HALF], all_embed[:, HALF:]])

    pad = E_PAD - N_EDGES
    row = jnp.concatenate([edge_index[0], jnp.zeros((pad,), edge_index.dtype)])
    col = jnp.concatenate([edge_index[1], jnp.zeros((pad,), edge_index.dtype)])
    val = jnp.concatenate([edge_vals, jnp.zeros((pad,), edge_vals.dtype)])
    packed = jnp.stack(
        [row.reshape(-1, 128), col.reshape(-1, 128)], axis=1,
    )
    vals2d = val.reshape(-1, 128)
    zeros = jnp.zeros((ROWS_PER_TILE, HALF), jnp.float32)

    tabs = [tab]
    for _ in range(N_HOPS):
        tabs.append(_hop(tabs[-1], packed, vals2d, zeros))

    embs = jnp.stack(
        [jnp.concatenate([t[0, :N_TOTAL], t[1, :N_TOTAL]], axis=-1) for t in tabs],
        axis=1,
    )
    return embs[:N_USERS], embs[N_USERS:]


# speedup vs baseline: 1.2770x; 1.2770x over previous
import functools

import jax
import jax.numpy as jnp
from jax import lax
from jax.experimental import pallas as pl
from jax.experimental.pallas import tpu as pltpu
from jax.experimental.pallas import tpu_sc as plsc

N_USERS = 50000
N_ITEMS = 50000
N_TOTAL = N_USERS + N_ITEMS
EMB_DIM = 32
HALF = 16
N_EDGES = 1600000
N_HOPS = 3

NS = 16
K = 8
CHUNK = K * 128
STAGES = 98
EDGES_PER_TILE = STAGES * CHUNK
E_PAD = NS * EDGES_PER_TILE
N_PAD = 100096
ROWS_PER_TILE = N_PAD // NS

_mesh = plsc.VectorSubcoreMesh(core_axis_name="c", subcore_axis_name="s")


@functools.partial(
    pl.kernel,
    mesh=_mesh,
    out_type=jax.ShapeDtypeStruct((2, N_PAD, HALF), jnp.float32),
    compiler_params=pltpu.CompilerParams(use_tc_tiling_on_sc=False),
    scratch_types=[
        pltpu.VMEM((K, 128), jnp.int32),
        pltpu.VMEM((K, 128), jnp.int32),
        pltpu.VMEM((K, 128), jnp.float32),
        pltpu.VMEM((K, 128), jnp.int32),
        pltpu.VMEM((K, 128), jnp.int32),
        pltpu.VMEM((K, 128), jnp.float32),
        pltpu.VMEM((128, HALF), jnp.float32),
        pltpu.VMEM((128, HALF), jnp.float32),
        pltpu.SemaphoreType.DMA,
        pltpu.SemaphoreType.DMA,
        pltpu.SemaphoreType.DMA,
        pltpu.VMEM_SHARED((N_PAD, HALF), jnp.float32),
    ],
)
def _hop(tab_hbm, row_hbm, col_hbm, val_hbm, zeros_hbm, out_hbm,
         row_a, col_a, val_a, row_b, col_b, val_b, msg_a, msg_b,
         esem, gsem, ssem, acc_sh):
    c = lax.axis_index("c")
    s = lax.axis_index("s")
    tab = tab_hbm.at[c]

    pltpu.sync_copy(zeros_hbm, acc_sh.at[pl.ds(s * ROWS_PER_TILE, ROWS_PER_TILE)])
    plsc.subcore_barrier()

    base128 = s * (STAGES * K)

    def issue_edges(st, bufs):
        row_r, col_r, val_r = bufs
        pltpu.async_copy(row_hbm.at[pl.ds(st, K)], row_r, esem)
        pltpu.async_copy(col_hbm.at[pl.ds(st, K)], col_r, esem)
        pltpu.async_copy(val_hbm.at[pl.ds(st, K)], val_r, esem)

    def drain_edges(st, bufs):
        row_r, col_r, val_r = bufs
        pltpu.make_async_copy(row_hbm.at[pl.ds(st, K)], row_r, esem).wait()
        pltpu.make_async_copy(col_hbm.at[pl.ds(st, K)], col_r, esem).wait()
        pltpu.make_async_copy(val_hbm.at[pl.ds(st, K)], val_r, esem).wait()

    def scale(mb, val_r, j):
        def scale_group(g, _):
            vv = val_r[j, pl.ds(g * 16, 16)]
            base = g * 16
            for e in range(16):
                mb[base + e, :] = mb[base + e, :] * vv[e]
            return 0

        lax.fori_loop(0, 8, scale_group, 0)

    def stage_block(bufs):
        row_r, col_r, val_r = bufs
        mbs = (msg_a, msg_b)
        gather_h = [None, None]
        scatter_h = []
        gather_h[0] = pltpu.async_copy(tab.at[col_r.at[0]], mbs[0], gsem)
        for j in range(K):
            mb = mbs[j % 2]
            if j + 1 < K:
                if len(scatter_h) > 1:
                    scatter_h.pop(0).wait()
                gather_h[(j + 1) % 2] = pltpu.async_copy(
                    tab.at[col_r.at[j + 1]], mbs[(j + 1) % 2], gsem)
            gather_h[j % 2].wait()
            scale(mb, val_r, j)
            scatter_h.append(
                pltpu.async_copy(mb, acc_sh.at[row_r.at[j]], ssem, add=True))
        for h in scatter_h:
            h.wait()

    bufs_a = (row_a, col_a, val_a)
    bufs_b = (row_b, col_b, val_b)

    issue_edges(base128, bufs_a)

    def pair_body(t, _):
        st0 = base128 + (2 * t) * K
        st1 = st0 + K
        st2 = st1 + K
        drain_edges(st0, bufs_a)
        issue_edges(st1, bufs_b)
        stage_block(bufs_a)
        drain_edges(st1, bufs_b)

        @pl.when(t + 1 < STAGES // 2)
        def _():
            issue_edges(st2, bufs_a)

        stage_block(bufs_b)
        return 0

    lax.fori_loop(0, STAGES // 2, pair_body, 0)

    plsc.subcore_barrier()
    sl = pl.ds(s * ROWS_PER_TILE, ROWS_PER_TILE)
    pltpu.sync_copy(acc_sh.at[sl], out_hbm.at[c, sl])


def kernel(user_embed, item_embed, edge_index, edge_vals):
    all_embed = jnp.concatenate([user_embed, item_embed], axis=0)
    all_embed = jnp.pad(all_embed, ((0, N_PAD - N_TOTAL), (0, 0)))
    tab = jnp.stack([all_embed[:, :HALF], all_embed[:, HALF:]])

    pad = E_PAD - N_EDGES
    row = jnp.concatenate([edge_index[0], jnp.zeros((pad,), edge_index.dtype)])
    col = jnp.concatenate([edge_index[1], jnp.zeros((pad,), edge_index.dtype)])
    val = jnp.concatenate([edge_vals, jnp.zeros((pad,), edge_vals.dtype)])
    row = row.reshape(-1, 128)
    col = col.reshape(-1, 128)
    val = val.reshape(-1, 128)
    zeros = jnp.zeros((ROWS_PER_TILE, HALF), jnp.float32)

    tabs = [tab]
    for _ in range(N_HOPS):
        tabs.append(_hop(tabs[-1], row, col, val, zeros))

    embs = jnp.stack(
        [jnp.concatenate([t[0, :N_TOTAL], t[1, :N_TOTAL]], axis=-1) for t in tabs],
        axis=1,
    )
    return embs[:N_USERS], embs[N_USERS:]
